# baseline (speedup 1.0000x reference)
import functools

import jax
import jax.numpy as jnp
import numpy as np
from jax.experimental import pallas as pl
from jax.experimental.pallas import tpu as pltpu

MOD_F = 109
OUT_F = 128
K_FIX = 6
MOD_HID = 8
CHARGE_DIM = 2
AA_DIM = OUT_F - MOD_HID - CHARGE_DIM
L_SEQ = 200
BB = 32


def _body(idx_ref, mod_ref, chg_ref, A_ref, M_ref, pe_ref, out_ref):
    idx = idx_ref[...]
    mod = mod_ref[...]
    bb, l = idx.shape
    lane = jax.lax.broadcasted_iota(jnp.int32, (bb, l, 128), 2)
    oh = (idx[:, :, None] == lane).astype(jnp.bfloat16)
    x = jnp.dot(oh.reshape(bb * l, 128), A_ref[...].astype(jnp.bfloat16),
                preferred_element_type=jnp.float32)
    m = jnp.dot(mod.reshape(bb * l, MOD_F).astype(jnp.bfloat16),
                M_ref[...].astype(jnp.bfloat16),
                preferred_element_type=jnp.float32)
    out = x.reshape(bb, l, OUT_F) + m.reshape(bb, l, OUT_F) + pe_ref[...][None]
    clane = jax.lax.broadcasted_iota(jnp.int32, (1, 1, OUT_F), 2)
    chg = chg_ref[...].reshape(bb, 1, 1)
    out_ref[...] = out + jnp.where(clane >= OUT_F - CHARGE_DIM, chg, 0.0)


@jax.jit
def kernel(aa_indices, mod_x, charges, aa_table, W_mod, pe):
    B, L = aa_indices.shape
    A = jnp.zeros((128, OUT_F), jnp.float32).at[:, :AA_DIM].set(aa_table)
    M = jnp.zeros((MOD_F, OUT_F), jnp.float32)
    M = M.at[:K_FIX, AA_DIM:AA_DIM + K_FIX].set(jnp.eye(K_FIX, dtype=jnp.float32))
    M = M.at[K_FIX:, AA_DIM + K_FIX:AA_DIM + MOD_HID].set(W_mod.T)
    pe2d = pe[0, :L, :]

    grid = (B // BB,)
    out = pl.pallas_call(
        _body,
        grid=grid,
        in_specs=[
            pl.BlockSpec((BB, L), lambda i: (i, 0)),
            pl.BlockSpec((BB, L, MOD_F), lambda i: (i, 0, 0)),
            pl.BlockSpec((BB, 1), lambda i: (i, 0)),
            pl.BlockSpec((128, OUT_F), lambda i: (0, 0)),
            pl.BlockSpec((MOD_F, OUT_F), lambda i: (0, 0)),
            pl.BlockSpec((L, OUT_F), lambda i: (0, 0)),
        ],
        out_specs=pl.BlockSpec((BB, L, OUT_F), lambda i: (i, 0, 0)),
        out_shape=jax.ShapeDtypeStruct((B, L, OUT_F), jnp.float32),
        compiler_params=pltpu.CompilerParams(
            dimension_semantics=("parallel",),
        ),
    )(aa_indices, mod_x, charges, A, M, pe2d)
    return out


# speedup vs baseline: 1.0462x; 1.0462x over previous
import functools

import jax
import jax.numpy as jnp
import numpy as np
from jax.experimental import pallas as pl
from jax.experimental.pallas import tpu as pltpu

MOD_F = 109
OUT_F = 128
K_FIX = 6
MOD_HID = 8
CHARGE_DIM = 2
AA_DIM = OUT_F - MOD_HID - CHARGE_DIM
L_SEQ = 200
BB = 64


def _body(idx_ref, mod_ref, chg_ref, A_ref, M_ref, pe_ref, out_ref):
    idx = idx_ref[...]
    mod = mod_ref[...]
    bb, l = idx.shape
    lane = jax.lax.broadcasted_iota(jnp.int32, (bb, l, 128), 2)
    oh = (idx[:, :, None] == lane).astype(jnp.bfloat16)
    x = jnp.dot(oh.reshape(bb * l, 128), A_ref[...].astype(jnp.bfloat16),
                preferred_element_type=jnp.float32)
    m = jnp.dot(mod.reshape(bb * l, MOD_F).astype(jnp.bfloat16),
                M_ref[...].astype(jnp.bfloat16),
                preferred_element_type=jnp.float32)
    out = x.reshape(bb, l, OUT_F) + m.reshape(bb, l, OUT_F) + pe_ref[...][None]
    clane = jax.lax.broadcasted_iota(jnp.int32, (1, 1, OUT_F), 2)
    chg = chg_ref[...].reshape(bb, 1, 1)
    out_ref[...] = out + jnp.where(clane >= OUT_F - CHARGE_DIM, chg, 0.0)


@jax.jit
def kernel(aa_indices, mod_x, charges, aa_table, W_mod, pe):
    B, L = aa_indices.shape
    A = jnp.zeros((128, OUT_F), jnp.float32).at[:, :AA_DIM].set(aa_table)
    M = jnp.zeros((MOD_F, OUT_F), jnp.float32)
    M = M.at[:K_FIX, AA_DIM:AA_DIM + K_FIX].set(jnp.eye(K_FIX, dtype=jnp.float32))
    M = M.at[K_FIX:, AA_DIM + K_FIX:AA_DIM + MOD_HID].set(W_mod.T)
    pe2d = pe[0, :L, :]

    grid = (B // BB,)
    out = pl.pallas_call(
        _body,
        grid=grid,
        in_specs=[
            pl.BlockSpec((BB, L), lambda i: (i, 0)),
            pl.BlockSpec((BB, L, MOD_F), lambda i: (i, 0, 0)),
            pl.BlockSpec((BB, 1), lambda i: (i, 0)),
            pl.BlockSpec((128, OUT_F), lambda i: (0, 0)),
            pl.BlockSpec((MOD_F, OUT_F), lambda i: (0, 0)),
            pl.BlockSpec((L, OUT_F), lambda i: (0, 0)),
        ],
        out_specs=pl.BlockSpec((BB, L, OUT_F), lambda i: (i, 0, 0)),
        out_shape=jax.ShapeDtypeStruct((B, L, OUT_F), jnp.float32),
        compiler_params=pltpu.CompilerParams(
            dimension_semantics=("parallel",),
        ),
    )(aa_indices, mod_x, charges, A, M, pe2d)
    return out


# speedup vs baseline: 1.0488x; 1.0025x over previous
import functools

import jax
import jax.numpy as jnp
import numpy as np
from jax.experimental import pallas as pl
from jax.experimental.pallas import tpu as pltpu

MOD_F = 109
OUT_F = 128
K_FIX = 6
MOD_HID = 8
CHARGE_DIM = 2
AA_DIM = OUT_F - MOD_HID - CHARGE_DIM
L_SEQ = 200
BB = 64


def _body(idx_ref, mod_ref, chg_ref, A_ref, M_ref, pe_ref, out_ref):
    idx = idx_ref[...]
    mod = mod_ref[...]
    bb, l = idx.shape
    lane = jax.lax.broadcasted_iota(jnp.int32, (bb, l, 128), 2)
    oh = (idx[:, :, None] == lane).astype(jnp.bfloat16)
    x = jnp.dot(oh.reshape(bb * l, 128), A_ref[...],
                preferred_element_type=jnp.float32)
    m = jnp.dot(mod.reshape(bb * l, MOD_F).astype(jnp.bfloat16), M_ref[...],
                preferred_element_type=jnp.float32)
    out = x.reshape(bb, l, OUT_F) + m.reshape(bb, l, OUT_F) + pe_ref[...][None]
    clane = jax.lax.broadcasted_iota(jnp.int32, (1, 1, OUT_F), 2)
    chg = chg_ref[...].reshape(bb, 1, 1)
    out_ref[...] = out + jnp.where(clane >= OUT_F - CHARGE_DIM, chg, 0.0)


@jax.jit
def kernel(aa_indices, mod_x, charges, aa_table, W_mod, pe):
    B, L = aa_indices.shape
    A = jnp.zeros((128, OUT_F), jnp.float32).at[:, :AA_DIM].set(aa_table)
    M = jnp.zeros((MOD_F, OUT_F), jnp.float32)
    M = M.at[:K_FIX, AA_DIM:AA_DIM + K_FIX].set(jnp.eye(K_FIX, dtype=jnp.float32))
    M = M.at[K_FIX:, AA_DIM + K_FIX:AA_DIM + MOD_HID].set(W_mod.T)
    A = A.astype(jnp.bfloat16)
    M = M.astype(jnp.bfloat16)
    pe2d = pe[0, :L, :]

    grid = (B // BB,)
    out = pl.pallas_call(
        _body,
        grid=grid,
        in_specs=[
            pl.BlockSpec((BB, L), lambda i: (i, 0)),
            pl.BlockSpec((BB, L, MOD_F), lambda i: (i, 0, 0)),
            pl.BlockSpec((BB, 1), lambda i: (i, 0)),
            pl.BlockSpec((128, OUT_F), lambda i: (0, 0)),
            pl.BlockSpec((MOD_F, OUT_F), lambda i: (0, 0)),
            pl.BlockSpec((L, OUT_F), lambda i: (0, 0)),
        ],
        out_specs=pl.BlockSpec((BB, L, OUT_F), lambda i: (i, 0, 0)),
        out_shape=jax.ShapeDtypeStruct((B, L, OUT_F), jnp.float32),
        compiler_params=pltpu.CompilerParams(
            dimension_semantics=("parallel",),
        ),
    )(aa_indices, mod_x, charges, A, M, pe2d)
    return out
